# Initial kernel scaffold; baseline (speedup 1.0000x reference)
import jax
import jax.numpy as jnp
from jax.experimental import pallas as pl


def kernel(pred_logits, pred_boxes, proj_queries, target_sizes):
    raise NotImplementedError("write your pallas kernel here")


# speedup vs baseline: 2.4308x; 2.4308x over previous
import functools

import jax
import jax.numpy as jnp
from jax import lax
from jax.experimental import pallas as pl
from jax.experimental.pallas import tpu as pltpu
from jax.experimental.pallas import tpu_sc as plsc

V = 30523
NQ = 900
NPAD = 1024
BQ = 60
GRID1 = NQ // BQ
K = 100
KPAD = 112


def _stage1_body(scale_ref, logits_ref, boxes_ref,
                 scores_ref, plast_ref, wl_ref, boxes_out_ref):
    x = logits_ref[0]
    xnl = x[:, :V - 1]
    m_nl = jnp.max(xnl, axis=-1, keepdims=True)
    amax = jnp.argmax(xnl, axis=-1)[:, None]
    l_last = x[:, V - 1:V]
    m_all = jnp.maximum(m_nl, l_last)
    z = jnp.sum(jnp.exp(x - m_all), axis=-1, keepdims=True)
    scores_ref[0] = jnp.exp(m_nl - m_all) / z
    plast_ref[0] = jnp.exp(l_last - m_all) / z
    wl_ref[0] = amax
    b = boxes_ref[0]
    cx, cy, w, h = b[:, 0:1], b[:, 1:2], b[:, 2:3], b[:, 3:4]
    xyxy = jnp.concatenate(
        [cx - 0.5 * w, cy - 0.5 * h, cx + 0.5 * w, cy + 0.5 * h], axis=-1)
    boxes_out_ref[0] = xyxy * scale_ref[...]


def _stage1(logits3d, boxes3d, scale):
    return pl.pallas_call(
        _stage1_body,
        grid=(GRID1,),
        in_specs=[
            pl.BlockSpec((1, 4), lambda i: (0, 0)),
            pl.BlockSpec((1, BQ, V), lambda i: (i, 0, 0)),
            pl.BlockSpec((1, BQ, 4), lambda i: (i, 0, 0)),
        ],
        out_specs=[
            pl.BlockSpec((1, BQ, 1), lambda i: (i, 0, 0)),
            pl.BlockSpec((1, BQ, 1), lambda i: (i, 0, 0)),
            pl.BlockSpec((1, BQ, 1), lambda i: (i, 0, 0)),
            pl.BlockSpec((1, BQ, 4), lambda i: (i, 0, 0)),
        ],
        out_shape=[
            jax.ShapeDtypeStruct((GRID1, BQ, 1), jnp.float32),
            jax.ShapeDtypeStruct((GRID1, BQ, 1), jnp.float32),
            jax.ShapeDtypeStruct((GRID1, BQ, 1), jnp.int32),
            jax.ShapeDtypeStruct((GRID1, BQ, 4), jnp.float32),
        ],
    )(scale, logits3d, boxes3d)


def kernel(pred_logits, pred_boxes, proj_queries, target_sizes):
    logits3d = pred_logits.reshape(GRID1, BQ, V)
    boxes3d = pred_boxes.reshape(GRID1, BQ, 4)
    img_h = target_sizes[:, 0].astype(jnp.float32)
    img_w = target_sizes[:, 1].astype(jnp.float32)
    scale = jnp.stack([img_w, img_h, img_w, img_h], axis=1)

    scores_p, plast_p, wl_p, boxes_s = _stage1(logits3d, boxes3d, scale)

    sc900 = scores_p.reshape(NQ)
    topk_scores, topk_idx = lax.top_k(sc900, K)
    scores = (1.0 - plast_p.reshape(NQ)[topk_idx])[None]
    labels = jnp.zeros((1, K), jnp.float32)
    boxes = boxes_s.reshape(NQ, 4)[topk_idx][None]
    word_labels = wl_p.reshape(NQ)[topk_idx][None]
    proj_q = proj_queries[:, topk_idx]
    return (scores, labels, boxes, word_labels, proj_q)
